# baseline (speedup 1.0000x reference)
import functools

import jax
import jax.numpy as jnp
from jax import lax
from jax.experimental import pallas as pl
from jax.experimental.pallas import tpu as pltpu
from jax.experimental.pallas import tpu_sc as plsc

NC = 2
NS = 16
L = 16
NW = NC * NS

BATCH = 16384
EMB = 32
BPW = BATCH // NW
PH = 64
NPH = BPW // PH
HC = 32
BLINES = 7813


def _mf_body(user_hbm, item_hbm, ue_hbm, ie_hbm, ub_hbm, ib_hbm, out_hbm,
             idx_u, idx_i, iu2a, iu2b, ii2a, ii2b, iu7a, iu7b, ii7a, ii7b,
             eu_a, eu_b, ei_a, ei_b, bu_a, bu_b, bi_a, bi_b, outv,
             sem_eu, sem_ei, sem_bu, sem_bi):
    wid = lax.axis_index("s") * NC + lax.axis_index("c")
    base = wid * BPW

    lane = lax.iota(jnp.int32, L)

    for ph in range(NPH):
        off = base + ph * PH
        pltpu.sync_copy(user_hbm.at[pl.ds(off, PH)], idx_u)
        pltpu.sync_copy(item_hbm.at[pl.ds(off, PH)], idx_i)

        for k in range(PH // L):
            vu = idx_u[pl.ds(k * L, L)]
            vi = idx_i[pl.ds(k * L, L)]
            h = k % (HC // L)
            ut2 = iu2a if k < HC // L else iu2b
            it2 = ii2a if k < HC // L else ii2b
            ut7 = iu7a if k < HC // L else iu7b
            it7 = ii7a if k < HC // L else ii7b
            ut2[pl.ds(h * L, L)] = lax.shift_right_logical(vu, 2)
            it2[pl.ds(h * L, L)] = lax.shift_right_logical(vi, 2)
            ut7[pl.ds(h * L, L)] = lax.shift_right_logical(vu, 7)
            it7[pl.ds(h * L, L)] = lax.shift_right_logical(vi, 7)

        copies = [
            pltpu.make_async_copy(ue_hbm.at[iu2a], eu_a, sem_eu),
            pltpu.make_async_copy(ue_hbm.at[iu2b], eu_b, sem_eu),
            pltpu.make_async_copy(ie_hbm.at[ii2a], ei_a, sem_ei),
            pltpu.make_async_copy(ie_hbm.at[ii2b], ei_b, sem_ei),
            pltpu.make_async_copy(ub_hbm.at[iu7a], bu_a, sem_bu),
            pltpu.make_async_copy(ub_hbm.at[iu7b], bu_b, sem_bu),
            pltpu.make_async_copy(ib_hbm.at[ii7a], bi_a, sem_bi),
            pltpu.make_async_copy(ib_hbm.at[ii7b], bi_b, sem_bi),
        ]
        for c in copies:
            c.start()
        for c in copies:
            c.wait()

        def make_body(eu, ei, bu, bi, goff):
            def g_body(g, carry):
                rows = g * L + lane
                vu = idx_u[pl.ds((goff + g) * L, L)]
                vi = idx_i[pl.ds((goff + g) * L, L)]
                acc = plsc.load_gather(bu, [rows, vu & 127])
                acc = acc + plsc.load_gather(bi, [rows, vi & 127])
                ucb = (vu & 3) * EMB
                icb = (vi & 3) * EMB
                for d in range(EMB):
                    u = plsc.load_gather(eu, [rows, ucb + d])
                    v = plsc.load_gather(ei, [rows, icb + d])
                    acc = acc + u * v
                outv[pl.ds(ph * PH + (goff + g) * L, L)] = acc
                return carry
            return g_body

        lax.fori_loop(0, HC // L, make_body(eu_a, ei_a, bu_a, bi_a, 0), 0)
        lax.fori_loop(0, HC // L, make_body(eu_b, ei_b, bu_b, bi_b, HC // L), 0)

    pltpu.sync_copy(outv, out_hbm.at[pl.ds(base, BPW)])


@functools.partial(jax.jit, static_argnums=())
def _mf_call(user, item, ue4, ie4, ubp, ibp):
    mesh = plsc.VectorSubcoreMesh(core_axis_name="c", subcore_axis_name="s")
    run = pl.kernel(
        _mf_body,
        out_type=jax.ShapeDtypeStruct((BATCH,), jnp.float32),
        mesh=mesh,
        compiler_params=pltpu.CompilerParams(needs_layout_passes=False),
        scratch_types=[
            pltpu.VMEM((PH,), jnp.int32),
            pltpu.VMEM((PH,), jnp.int32),
            pltpu.VMEM((HC,), jnp.int32),
            pltpu.VMEM((HC,), jnp.int32),
            pltpu.VMEM((HC,), jnp.int32),
            pltpu.VMEM((HC,), jnp.int32),
            pltpu.VMEM((HC,), jnp.int32),
            pltpu.VMEM((HC,), jnp.int32),
            pltpu.VMEM((HC,), jnp.int32),
            pltpu.VMEM((HC,), jnp.int32),
            pltpu.VMEM((HC, 128), jnp.float32),
            pltpu.VMEM((HC, 128), jnp.float32),
            pltpu.VMEM((HC, 128), jnp.float32),
            pltpu.VMEM((HC, 128), jnp.float32),
            pltpu.VMEM((HC, 128), jnp.float32),
            pltpu.VMEM((HC, 128), jnp.float32),
            pltpu.VMEM((HC, 128), jnp.float32),
            pltpu.VMEM((HC, 128), jnp.float32),
            pltpu.VMEM((BPW,), jnp.float32),
            pltpu.SemaphoreType.DMA,
            pltpu.SemaphoreType.DMA,
            pltpu.SemaphoreType.DMA,
            pltpu.SemaphoreType.DMA,
        ],
    )
    return run(user, item, ue4, ie4, ubp, ibp)


def kernel(user, item, user_emb_w, item_emb_w, user_bias_w, item_bias_w):
    user = user.astype(jnp.int32)
    item = item.astype(jnp.int32)
    ue4 = user_emb_w.reshape(-1, 128)
    ie4 = item_emb_w.reshape(-1, 128)
    pad = BLINES * 128 - user_bias_w.size
    ubp = jnp.pad(user_bias_w.reshape(-1), (0, pad)).reshape(BLINES, 128)
    ibp = jnp.pad(item_bias_w.reshape(-1), (0, pad)).reshape(BLINES, 128)
    return _mf_call(user, item, ue4, ie4, ubp, ibp)
